# Initial kernel scaffold; baseline (speedup 1.0000x reference)
import functools

import jax
import jax.numpy as jnp
from jax import lax
from jax.experimental import pallas as pl
from jax.experimental.pallas import tpu as pltpu
from jax.experimental.pallas import tpu_sc as plsc

BATCH = 1024
MAX_LEN = 200
D_MODEL = 128
LANES = 16

NC, NS = 2, 16
NW = NC * NS
TOK = BATCH * MAX_LEN
TPW = TOK // NW
CH = 100
NCHUNK = TPW // CH


def _pe():
    pos = jnp.arange(MAX_LEN, dtype=jnp.float32)[:, None]
    i = jnp.arange(0, D_MODEL, 2, dtype=jnp.float32)
    div = jnp.exp(-(jnp.log(10000.0) / D_MODEL) * i)
    ang = pos * div[None, :]
    pe = jnp.zeros((MAX_LEN, D_MODEL), dtype=jnp.float32)
    pe = pe.at[:, 0::2].set(jnp.sin(ang))
    pe = pe.at[:, 1::2].set(jnp.cos(ang))
    return pe


_mesh = plsc.VectorSubcoreMesh(core_axis_name="c", subcore_axis_name="s")


@functools.partial(
    pl.kernel,
    out_type=jax.ShapeDtypeStruct((TOK, D_MODEL), jnp.float32),
    mesh=_mesh,
    scratch_types=[
        pltpu.VMEM((NCHUNK, CH), jnp.int32),
        pltpu.VMEM((MAX_LEN, D_MODEL), jnp.float32),
        pltpu.VMEM((CH, D_MODEL), jnp.float32),
        pltpu.SemaphoreType.DMA,
    ],
)
def _sc_embed(table_hbm, idx_hbm, pe_hbm, out_hbm, idx_v, pe_v, rows_v, sem):
    wid = lax.axis_index("s") * NC + lax.axis_index("c")
    pltpu.sync_copy(idx_hbm.at[wid], idx_v)
    pltpu.sync_copy(pe_hbm, pe_v)
    base = wid * TPW

    @pl.loop(0, NCHUNK)
    def chunk_body(c):
        pltpu.async_copy(table_hbm.at[idx_v.at[c]], rows_v, sem).wait()
        poff = (c % 2) * CH

        @pl.loop(0, CH)
        def row_body(r):
            p = poff + r
            for j in range(D_MODEL // LANES):
                sl = pl.ds(j * LANES, LANES)
                plsc.addupdate(rows_v.at[r, sl], pe_v[p, sl])

        pltpu.sync_copy(rows_v, out_hbm.at[pl.ds(base + c * CH, CH)])


def kernel(batch, table):
    idx = batch.astype(jnp.int32).reshape(NW, NCHUNK, CH)
    out = _sc_embed(table, idx, _pe())
    return out.reshape(BATCH, MAX_LEN, D_MODEL)


# speedup vs baseline: 1.6172x; 1.6172x over previous
import functools

import jax
import jax.numpy as jnp
from jax import lax
from jax.experimental import pallas as pl
from jax.experimental.pallas import tpu as pltpu
from jax.experimental.pallas import tpu_sc as plsc

BATCH = 1024
MAX_LEN = 200
D_MODEL = 128
LANES = 16

NC, NS = 2, 16
NW = NC * NS
TOK = BATCH * MAX_LEN
TPW = TOK // NW
CH = 40
NCHUNK = TPW // CH


def _pe():
    pos = jnp.arange(MAX_LEN, dtype=jnp.float32)[:, None]
    i = jnp.arange(0, D_MODEL, 2, dtype=jnp.float32)
    div = jnp.exp(-(jnp.log(10000.0) / D_MODEL) * i)
    ang = pos * div[None, :]
    pe = jnp.zeros((MAX_LEN, D_MODEL), dtype=jnp.float32)
    pe = pe.at[:, 0::2].set(jnp.sin(ang))
    pe = pe.at[:, 1::2].set(jnp.cos(ang))
    return pe


_mesh = plsc.VectorSubcoreMesh(core_axis_name="c", subcore_axis_name="s")


@functools.partial(
    pl.kernel,
    out_type=jax.ShapeDtypeStruct((TOK, D_MODEL), jnp.float32),
    mesh=_mesh,
    scratch_types=[
        pltpu.VMEM((NCHUNK, CH), jnp.int32),
        pltpu.VMEM((MAX_LEN, D_MODEL), jnp.float32),
        pltpu.VMEM((CH, D_MODEL), jnp.float32),
        pltpu.SemaphoreType.DMA,
    ],
)
def _sc_embed(table_hbm, idx_hbm, pe_hbm, out_hbm, idx_v, pe_v, rows_v, sem):
    wid = lax.axis_index("s") * NC + lax.axis_index("c")
    pltpu.sync_copy(idx_hbm.at[wid], idx_v)
    pltpu.sync_copy(pe_hbm, pe_v)
    base = wid * TPW

    @pl.loop(0, NCHUNK)
    def chunk_body(c):
        pltpu.async_copy(table_hbm.at[idx_v.at[c]], rows_v, sem).wait()
        poff = (c % (MAX_LEN // CH)) * CH

        @pl.loop(0, CH)
        def row_body(r):
            p = poff + r
            for j in range(D_MODEL // LANES):
                sl = pl.ds(j * LANES, LANES)
                plsc.addupdate(rows_v.at[r, sl], pe_v[p, sl])

        pltpu.sync_copy(rows_v, out_hbm.at[pl.ds(base + c * CH, CH)])


def kernel(batch, table):
    idx = batch.astype(jnp.int32).reshape(NW, NCHUNK, CH)
    out = _sc_embed(table, idx, _pe())
    return out.reshape(BATCH, MAX_LEN, D_MODEL)


# speedup vs baseline: 2.4555x; 1.5184x over previous
import functools

import jax
import jax.numpy as jnp
from jax import lax
from jax.experimental import pallas as pl
from jax.experimental.pallas import tpu as pltpu
from jax.experimental.pallas import tpu_sc as plsc

BATCH = 1024
MAX_LEN = 200
D_MODEL = 128
LANES = 16

NC, NS = 2, 16
NW = NC * NS
TOK = BATCH * MAX_LEN
TPW = TOK // NW
CH = 40
NCHUNK = TPW // CH
NBUF = 8
NGROUP = NCHUNK // NBUF


def _pe():
    pos = jnp.arange(MAX_LEN, dtype=jnp.float32)[:, None]
    i = jnp.arange(0, D_MODEL, 2, dtype=jnp.float32)
    div = jnp.exp(-(jnp.log(10000.0) / D_MODEL) * i)
    ang = pos * div[None, :]
    pe = jnp.zeros((MAX_LEN, D_MODEL), dtype=jnp.float32)
    pe = pe.at[:, 0::2].set(jnp.sin(ang))
    pe = pe.at[:, 1::2].set(jnp.cos(ang))
    return pe


_mesh = plsc.VectorSubcoreMesh(core_axis_name="c", subcore_axis_name="s")


@functools.partial(
    pl.kernel,
    out_type=jax.ShapeDtypeStruct((TOK, D_MODEL), jnp.float32),
    mesh=_mesh,
    scratch_types=[
        pltpu.VMEM((NCHUNK, CH), jnp.int32),
        pltpu.VMEM((MAX_LEN, D_MODEL), jnp.float32),
        pltpu.VMEM((NBUF, CH, D_MODEL), jnp.float32),
        pltpu.SemaphoreType.DMA((NBUF,)),
        pltpu.SemaphoreType.DMA((NBUF,)),
    ],
)
def _sc_embed(table_hbm, idx_hbm, pe_hbm, out_hbm, idx_v, pe_v, rows_v, gsem, ssem):
    wid = lax.axis_index("s") * NC + lax.axis_index("c")
    pltpu.sync_copy(idx_hbm.at[wid], idx_v)
    pltpu.sync_copy(pe_hbm, pe_v)
    base = wid * TPW

    def out_slice(c):
        off = pl.multiple_of(base + c * CH, 8)
        return out_hbm.at[pl.ds(off, CH)]

    def fire_gather(c, b):
        return pltpu.async_copy(table_hbm.at[idx_v.at[c]], rows_v.at[b], gsem.at[b])

    def add_pe(c, b):
        poff = lax.rem(c, MAX_LEN // CH) * CH

        @pl.loop(0, CH, unroll=4)
        def row_body(r):
            p = poff + r
            for j in range(D_MODEL // LANES):
                sl = pl.ds(j * LANES, LANES)
                plsc.addupdate(rows_v.at[b, r, sl], pe_v[p, sl])

    def finish(c, b, gd):
        gd.wait()
        add_pe(c, b)
        return pltpu.async_copy(rows_v.at[b], out_slice(c), ssem.at[b])

    gds = [fire_gather(b, b) for b in range(NBUF)]
    for b in range(NBUF):
        finish(b, b, gds[b])

    @pl.loop(1, NGROUP)
    def group_body(g):
        c0 = g * NBUF
        gds = []
        for b in range(NBUF):
            pltpu.make_async_copy(rows_v.at[b], out_slice(c0 + b), ssem.at[b]).wait()
            gds.append(fire_gather(c0 + b, b))
        for b in range(NBUF):
            finish(c0 + b, b, gds[b])

    for b in range(NBUF):
        pltpu.make_async_copy(
            rows_v.at[b], out_slice((NGROUP - 1) * NBUF + b), ssem.at[b]
        ).wait()


def kernel(batch, table):
    idx = batch.astype(jnp.int32).reshape(NW, NCHUNK, CH)
    out = _sc_embed(table, idx, _pe())
    return out.reshape(BATCH, MAX_LEN, D_MODEL)


# speedup vs baseline: 6.8582x; 2.7930x over previous
import functools

import jax
import jax.numpy as jnp
from jax import lax
from jax.experimental import pallas as pl
from jax.experimental.pallas import tpu as pltpu
from jax.experimental.pallas import tpu_sc as plsc

BATCH = 1024
MAX_LEN = 200
D_MODEL = 128
VOCAB = 128
LANES = 16

NC, NS = 2, 16
NW = NC * NS
TOK = BATCH * MAX_LEN
TPW = TOK // NW
CH = 128
NCHUNK = TPW // CH
NBUF = 5
NGROUP = NCHUNK // NBUF


def _pe():
    pos = jnp.arange(MAX_LEN, dtype=jnp.float32)[:, None]
    i = jnp.arange(0, D_MODEL, 2, dtype=jnp.float32)
    div = jnp.exp(-(jnp.log(10000.0) / D_MODEL) * i)
    ang = pos * div[None, :]
    pe = jnp.zeros((MAX_LEN, D_MODEL), dtype=jnp.float32)
    pe = pe.at[:, 0::2].set(jnp.sin(ang))
    pe = pe.at[:, 1::2].set(jnp.cos(ang))
    return pe


def _fuse_body(table_ref, pe_ref, out_ref):
    out_ref[...] = pe_ref[...][:, None, :] + table_ref[...][None, :, :]


_fuse = pl.pallas_call(
    _fuse_body,
    out_shape=jax.ShapeDtypeStruct((MAX_LEN, VOCAB, D_MODEL), jnp.float32),
)

_mesh = plsc.VectorSubcoreMesh(core_axis_name="c", subcore_axis_name="s")


@functools.partial(
    pl.kernel,
    out_type=jax.ShapeDtypeStruct((TOK, D_MODEL), jnp.float32),
    mesh=_mesh,
    scratch_types=[
        pltpu.VMEM((TPW,), jnp.int32),
        pltpu.VMEM((TPW,), jnp.int32),
        pltpu.VMEM((NBUF, CH, D_MODEL), jnp.float32),
        pltpu.SemaphoreType.DMA((NBUF,)),
        pltpu.SemaphoreType.DMA((NBUF,)),
    ],
)
def _sc_gather(fused_hbm, idx_hbm, offs_hbm, out_hbm, idx_v, offs_v, rows_v, gsem, ssem):
    wid = lax.axis_index("s") * NC + lax.axis_index("c")
    pltpu.sync_copy(idx_hbm.at[wid], idx_v)
    pltpu.sync_copy(offs_hbm, offs_v)
    base = wid * TPW

    @pl.loop(0, TPW // LANES, unroll=8)
    def add_off(i):
        sl = pl.ds(i * LANES, LANES)
        plsc.addupdate(idx_v.at[sl], offs_v[sl])

    def out_slice(c):
        off = pl.multiple_of(base + c * CH, 8)
        return out_hbm.at[pl.ds(off, CH)]

    def fire_gather(c, b):
        return pltpu.async_copy(
            fused_hbm.at[idx_v.at[pl.ds(c * CH, CH)]], rows_v.at[b], gsem.at[b]
        )

    gds = [fire_gather(b, b) for b in range(NBUF)]
    for b in range(NBUF):
        gds[b].wait()
        pltpu.async_copy(rows_v.at[b], out_slice(b), ssem.at[b])

    @pl.loop(1, NGROUP)
    def group_body(g):
        c0 = g * NBUF
        gds = []
        for b in range(NBUF):
            pltpu.make_async_copy(rows_v.at[b], out_slice(c0 + b), ssem.at[b]).wait()
            gds.append(fire_gather(c0 + b, b))
        for b in range(NBUF):
            gds[b].wait()
            pltpu.async_copy(rows_v.at[b], out_slice(c0 + b), ssem.at[b])

    for b in range(NBUF):
        pltpu.make_async_copy(
            rows_v.at[b], out_slice((NGROUP - 1) * NBUF + b), ssem.at[b]
        ).wait()


def kernel(batch, table):
    fused = _fuse(table, _pe()).reshape(MAX_LEN * VOCAB, D_MODEL)
    idx = batch.astype(jnp.int32).reshape(NW, TPW)
    offs = (jnp.tile(jnp.arange(MAX_LEN, dtype=jnp.int32), TPW // MAX_LEN) * VOCAB)
    out = _sc_gather(fused, idx, offs)
    return out.reshape(BATCH, MAX_LEN, D_MODEL)


# speedup vs baseline: 6.9393x; 1.0118x over previous
import functools

import jax
import jax.numpy as jnp
from jax import lax
from jax.experimental import pallas as pl
from jax.experimental.pallas import tpu as pltpu
from jax.experimental.pallas import tpu_sc as plsc

BATCH = 1024
MAX_LEN = 200
D_MODEL = 128
VOCAB = 128
LANES = 16

NC, NS = 2, 16
NW = NC * NS
TOK = BATCH * MAX_LEN
TPW = TOK // NW
CH = 64
NCHUNK = TPW // CH
NBUF = 10
NGROUP = NCHUNK // NBUF


def _pe():
    pos = jnp.arange(MAX_LEN, dtype=jnp.float32)[:, None]
    i = jnp.arange(0, D_MODEL, 2, dtype=jnp.float32)
    div = jnp.exp(-(jnp.log(10000.0) / D_MODEL) * i)
    ang = pos * div[None, :]
    pe = jnp.zeros((MAX_LEN, D_MODEL), dtype=jnp.float32)
    pe = pe.at[:, 0::2].set(jnp.sin(ang))
    pe = pe.at[:, 1::2].set(jnp.cos(ang))
    return pe


def _fuse_body(table_ref, pe_ref, out_ref):
    out_ref[...] = pe_ref[...][:, None, :] + table_ref[...][None, :, :]


_fuse = pl.pallas_call(
    _fuse_body,
    out_shape=jax.ShapeDtypeStruct((MAX_LEN, VOCAB, D_MODEL), jnp.float32),
)

_mesh = plsc.VectorSubcoreMesh(core_axis_name="c", subcore_axis_name="s")


@functools.partial(
    pl.kernel,
    out_type=jax.ShapeDtypeStruct((TOK, D_MODEL), jnp.float32),
    mesh=_mesh,
    scratch_types=[
        pltpu.VMEM((TPW,), jnp.int32),
        pltpu.VMEM((TPW,), jnp.int32),
        pltpu.VMEM((NBUF, CH, D_MODEL), jnp.float32),
        pltpu.SemaphoreType.DMA((NBUF,)),
        pltpu.SemaphoreType.DMA((NBUF,)),
    ],
)
def _sc_gather(fused_hbm, idx_hbm, offs_hbm, out_hbm, idx_v, offs_v, rows_v, gsem, ssem):
    wid = lax.axis_index("s") * NC + lax.axis_index("c")
    pltpu.sync_copy(idx_hbm.at[wid], idx_v)
    pltpu.sync_copy(offs_hbm, offs_v)
    base = wid * TPW

    @pl.loop(0, TPW // LANES, unroll=8)
    def add_off(i):
        sl = pl.ds(i * LANES, LANES)
        plsc.addupdate(idx_v.at[sl], offs_v[sl])

    def out_slice(c):
        off = pl.multiple_of(base + c * CH, 8)
        return out_hbm.at[pl.ds(off, CH)]

    def fire_gather(c, b):
        return pltpu.async_copy(
            fused_hbm.at[idx_v.at[pl.ds(c * CH, CH)]], rows_v.at[b], gsem.at[b]
        )

    gds = [fire_gather(b, b) for b in range(NBUF)]
    for b in range(NBUF):
        gds[b].wait()
        pltpu.async_copy(rows_v.at[b], out_slice(b), ssem.at[b])

    @pl.loop(1, NGROUP)
    def group_body(g):
        c0 = g * NBUF
        gds = []
        for b in range(NBUF):
            pltpu.make_async_copy(rows_v.at[b], out_slice(c0 + b), ssem.at[b]).wait()
            gds.append(fire_gather(c0 + b, b))
        for b in range(NBUF):
            gds[b].wait()
            pltpu.async_copy(rows_v.at[b], out_slice(c0 + b), ssem.at[b])

    for b in range(NBUF):
        pltpu.make_async_copy(
            rows_v.at[b], out_slice((NGROUP - 1) * NBUF + b), ssem.at[b]
        ).wait()


def kernel(batch, table):
    fused = _fuse(table, _pe()).reshape(MAX_LEN * VOCAB, D_MODEL)
    idx = batch.astype(jnp.int32).reshape(NW, TPW)
    offs = (jnp.tile(jnp.arange(MAX_LEN, dtype=jnp.int32), TPW // MAX_LEN) * VOCAB)
    out = _sc_gather(fused, idx, offs)
    return out.reshape(BATCH, MAX_LEN, D_MODEL)


# speedup vs baseline: 6.9963x; 1.0082x over previous
import functools

import jax
import jax.numpy as jnp
from jax import lax
from jax.experimental import pallas as pl
from jax.experimental.pallas import tpu as pltpu
from jax.experimental.pallas import tpu_sc as plsc

BATCH = 1024
MAX_LEN = 200
D_MODEL = 128
VOCAB = 128
LANES = 16

NC, NS = 2, 16
NW = NC * NS
TOK = BATCH * MAX_LEN
TPW = TOK // NW
CH = 128
NCHUNK = TPW // CH
NBUF = 5
NGROUP = NCHUNK // NBUF


def _pe():
    pos = jnp.arange(MAX_LEN, dtype=jnp.float32)[:, None]
    i = jnp.arange(0, D_MODEL, 2, dtype=jnp.float32)
    div = jnp.exp(-(jnp.log(10000.0) / D_MODEL) * i)
    ang = pos * div[None, :]
    pe = jnp.zeros((MAX_LEN, D_MODEL), dtype=jnp.float32)
    pe = pe.at[:, 0::2].set(jnp.sin(ang))
    pe = pe.at[:, 1::2].set(jnp.cos(ang))
    return pe


def _fuse_body(table_ref, pe_ref, idx_ref, fused_ref, idx2_ref):
    fused_ref[...] = pe_ref[...][:, None, :] + table_ref[...][None, :, :]
    t = lax.broadcasted_iota(jnp.int32, (NW, TPW), 1)
    idx2_ref[...] = idx_ref[...] + lax.rem(t, MAX_LEN) * VOCAB


_fuse = pl.pallas_call(
    _fuse_body,
    out_shape=(
        jax.ShapeDtypeStruct((MAX_LEN, VOCAB, D_MODEL), jnp.float32),
        jax.ShapeDtypeStruct((NW, TPW), jnp.int32),
    ),
)

_mesh = plsc.VectorSubcoreMesh(core_axis_name="c", subcore_axis_name="s")


@functools.partial(
    pl.kernel,
    out_type=jax.ShapeDtypeStruct((TOK, D_MODEL), jnp.float32),
    mesh=_mesh,
    scratch_types=[
        pltpu.VMEM((TPW,), jnp.int32),
        pltpu.VMEM((NBUF, CH, D_MODEL), jnp.float32),
        pltpu.SemaphoreType.DMA((NBUF,)),
        pltpu.SemaphoreType.DMA((NBUF,)),
    ],
)
def _sc_gather(fused_hbm, idx_hbm, out_hbm, idx_v, rows_v, gsem, ssem):
    wid = lax.axis_index("s") * NC + lax.axis_index("c")
    pltpu.sync_copy(idx_hbm.at[wid], idx_v)
    base = wid * TPW

    def out_slice(c):
        off = pl.multiple_of(base + c * CH, 8)
        return out_hbm.at[pl.ds(off, CH)]

    def fire_gather(c, b):
        return pltpu.async_copy(
            fused_hbm.at[idx_v.at[pl.ds(c * CH, CH)]], rows_v.at[b], gsem.at[b]
        )

    gds = [fire_gather(b, b) for b in range(NBUF)]
    for b in range(NBUF):
        gds[b].wait()
        pltpu.async_copy(rows_v.at[b], out_slice(b), ssem.at[b])

    @pl.loop(1, NGROUP)
    def group_body(g):
        c0 = g * NBUF
        gds = []
        for b in range(NBUF):
            pltpu.make_async_copy(rows_v.at[b], out_slice(c0 + b), ssem.at[b]).wait()
            gds.append(fire_gather(c0 + b, b))
        for b in range(NBUF):
            gds[b].wait()
            pltpu.async_copy(rows_v.at[b], out_slice(c0 + b), ssem.at[b])

    for b in range(NBUF):
        pltpu.make_async_copy(
            rows_v.at[b], out_slice((NGROUP - 1) * NBUF + b), ssem.at[b]
        ).wait()


def kernel(batch, table):
    idx = batch.astype(jnp.int32).reshape(NW, TPW)
    fused, idx2 = _fuse(table, _pe(), idx)
    out = _sc_gather(fused.reshape(MAX_LEN * VOCAB, D_MODEL), idx2)
    return out.reshape(BATCH, MAX_LEN, D_MODEL)


# speedup vs baseline: 7.0916x; 1.0136x over previous
import functools

import jax
import jax.numpy as jnp
from jax import lax
from jax.experimental import pallas as pl
from jax.experimental.pallas import tpu as pltpu
from jax.experimental.pallas import tpu_sc as plsc

BATCH = 1024
MAX_LEN = 200
D_MODEL = 128
VOCAB = 128
LANES = 16

NC, NS = 2, 16
NW = NC * NS
TOK = BATCH * MAX_LEN
TPW = TOK // NW
CH = 64
NCHUNK = TPW // CH
NBUF = 10
NGROUP = NCHUNK // NBUF


def _pe():
    pos = jnp.arange(MAX_LEN, dtype=jnp.float32)[:, None]
    i = jnp.arange(0, D_MODEL, 2, dtype=jnp.float32)
    div = jnp.exp(-(jnp.log(10000.0) / D_MODEL) * i)
    ang = pos * div[None, :]
    pe = jnp.zeros((MAX_LEN, D_MODEL), dtype=jnp.float32)
    pe = pe.at[:, 0::2].set(jnp.sin(ang))
    pe = pe.at[:, 1::2].set(jnp.cos(ang))
    return pe


def _fuse_body(table_ref, pe_ref, idx_ref, fused_ref, idx2_ref):
    fused_ref[...] = pe_ref[...][:, None, :] + table_ref[...][None, :, :]
    t = lax.broadcasted_iota(jnp.int32, (NW, TPW), 1)
    idx2_ref[...] = idx_ref[...] + lax.rem(t, MAX_LEN) * VOCAB


_fuse = pl.pallas_call(
    _fuse_body,
    out_shape=(
        jax.ShapeDtypeStruct((MAX_LEN, VOCAB, D_MODEL), jnp.float32),
        jax.ShapeDtypeStruct((NW, TPW), jnp.int32),
    ),
)

_mesh = plsc.VectorSubcoreMesh(core_axis_name="c", subcore_axis_name="s")


@functools.partial(
    pl.kernel,
    out_type=jax.ShapeDtypeStruct((TOK, D_MODEL), jnp.float32),
    mesh=_mesh,
    scratch_types=[
        pltpu.VMEM((TPW,), jnp.int32),
        pltpu.VMEM((NBUF, CH, D_MODEL), jnp.float32),
        pltpu.SemaphoreType.DMA((NBUF,)),
        pltpu.SemaphoreType.DMA((NBUF,)),
    ],
)
def _sc_gather(fused_hbm, idx_hbm, out_hbm, idx_v, rows_v, gsem, ssem):
    wid = lax.axis_index("s") * NC + lax.axis_index("c")
    pltpu.sync_copy(idx_hbm.at[wid], idx_v)
    base = wid * TPW

    def out_slice(c):
        off = pl.multiple_of(base + c * CH, 8)
        return out_hbm.at[pl.ds(off, CH)]

    def fire_gather(c, b):
        return pltpu.async_copy(
            fused_hbm.at[idx_v.at[pl.ds(c * CH, CH)]], rows_v.at[b], gsem.at[b]
        )

    gds = [fire_gather(b, b) for b in range(NBUF)]
    for b in range(NBUF):
        gds[b].wait()
        pltpu.async_copy(rows_v.at[b], out_slice(b), ssem.at[b])

    @pl.loop(1, NGROUP)
    def group_body(g):
        c0 = g * NBUF
        gds = []
        for b in range(NBUF):
            pltpu.make_async_copy(rows_v.at[b], out_slice(c0 + b), ssem.at[b]).wait()
            gds.append(fire_gather(c0 + b, b))
        for b in range(NBUF):
            gds[b].wait()
            pltpu.async_copy(rows_v.at[b], out_slice(c0 + b), ssem.at[b])

    for b in range(NBUF):
        pltpu.make_async_copy(
            rows_v.at[b], out_slice((NGROUP - 1) * NBUF + b), ssem.at[b]
        ).wait()


def kernel(batch, table):
    idx = batch.astype(jnp.int32).reshape(NW, TPW)
    fused, idx2 = _fuse(table, _pe(), idx)
    out = _sc_gather(fused.reshape(MAX_LEN * VOCAB, D_MODEL), idx2)
    return out.reshape(BATCH, MAX_LEN, D_MODEL)
